# baseline (device time: 15391 ns/iter reference)
import jax
import jax.numpy as jnp
from jax import lax
from jax.experimental import pallas as pl
from jax.experimental.pallas import tpu as pltpu

N_DEV = 4
B, Sq, Skv, Hq, Dh = 2, 256, 1024, 4, 64
HD = Hq * Dh
D = 512
S_SH = Skv // N_DEV
NCH = 2
SQ_C = Sq // NCH
NC = B * NCH
RC = SQ_C + 8
F32 = jnp.float32
BF16 = jnp.bfloat16


def kernel(x, Wq, K_ext, V_ext, Wo):
    x16 = x.astype(BF16)
    KV16 = jnp.concatenate(
        [K_ext.reshape(B, S_SH, HD), V_ext.reshape(B, S_SH, HD)],
        axis=-1).astype(BF16)

    def body(x_ref, wq_ref, kv_ref, wo_ref, out_ref,
             pbuf, abuf, abuf_bf, rbuf, csend, crecv):
        my = lax.axis_index("i")
        left = lax.rem(my + N_DEV - 1, N_DEV)
        right = lax.rem(my + 1, N_DEV)
        p1 = my ^ 1
        p2 = 3 - my

        barrier = pltpu.get_barrier_semaphore()
        for nbr in (left, right):
            pl.semaphore_signal(barrier, inc=1, device_id=(nbr,),
                                device_id_type=pl.DeviceIdType.MESH)

        koff = my * S_SH
        wq16 = wq_ref[...].astype(BF16)
        wo16 = wo_ref[...].astype(BF16)

        def rdma(stage, c, src, target):
            return pltpu.make_async_remote_copy(
                src_ref=src.at[c], dst_ref=rbuf.at[stage, c],
                send_sem=csend.at[stage, c], recv_sem=crecv.at[stage, c],
                device_id=(target,), device_id_type=pl.DeviceIdType.MESH)

        s1 = [None] * NC
        s2 = [None] * NC
        first = True
        for b in range(B):
            k16 = kv_ref[b, :, :HD]
            v16 = kv_ref[b, :, HD:]
            for half in range(NCH):
                c = b * NCH + half
                r0 = half * SQ_C
                qi = lax.broadcasted_iota(jnp.int32, (SQ_C, S_SH), 0) + r0
                kig = lax.broadcasted_iota(jnp.int32, (SQ_C, S_SH), 1) + koff
                mask = (jnp.abs(qi - kig) <= 128) | (kig < 32) | (qi < 32)

                q_c = jnp.dot(
                    x_ref[b, r0:r0 + SQ_C, :], wq16,
                    preferred_element_type=F32).astype(BF16)
                lcols = []
                for h in range(Hq):
                    qh = q_c[:, h * Dh:(h + 1) * Dh]
                    kh = k16[:, h * Dh:(h + 1) * Dh]
                    s = lax.dot_general(
                        qh, kh, (((1,), (1,)), ((), ())),
                        preferred_element_type=F32) * 0.125
                    w = jnp.where(mask, jnp.exp(s), 0.0)
                    vh = v16[:, h * Dh:(h + 1) * Dh]
                    pbuf[c, :SQ_C, h * Dh:(h + 1) * Dh] = jnp.dot(
                        w.astype(BF16), vh,
                        preferred_element_type=F32).astype(BF16)
                    lcols.append(jnp.sum(w, axis=1, keepdims=True))
                l_t = jnp.transpose(
                    jnp.concatenate(
                        lcols + [jnp.zeros((SQ_C, 8 - Hq), F32)],
                        axis=1))
                pbuf[c, SQ_C:, :] = jnp.concatenate(
                    [l_t, jnp.zeros((8, HD - SQ_C), F32)],
                    axis=1).astype(BF16)
                if first:
                    pl.semaphore_wait(barrier, 2)
                    first = False
                s1[c] = rdma(0, c, pbuf, p1)
                s1[c].start()

        for c in range(NC):
            s1[c].wait()
            abuf[c] = pbuf[c].astype(F32) + rbuf[0, c].astype(F32)
            abuf_bf[c] = abuf[c].astype(BF16)
            s2[c] = rdma(1, c, abuf_bf, p2)
            s2[c].start()

        for c in range(NC):
            b, half = divmod(c, NCH)
            s2[c].wait()
            tot = abuf[c] + rbuf[1, c].astype(F32)
            ctx = tot[:SQ_C, :]
            l_c = jnp.transpose(tot[SQ_C:, :SQ_C])
            rcp = 1.0 / l_c
            parts = []
            for h in range(Hq):
                parts.append(ctx[:, h * Dh:(h + 1) * Dh] * rcp[:, h:h + 1])
            ctx_n = jnp.concatenate(parts, axis=1)
            out_ref[b, half * SQ_C:(half + 1) * SQ_C, :] = jnp.dot(
                ctx_n.astype(BF16), wo16,
                preferred_element_type=F32).astype(BF16)

    return pl.pallas_call(
        body,
        out_shape=jax.ShapeDtypeStruct((B, Sq, D), BF16),
        in_specs=[pl.BlockSpec(memory_space=pltpu.VMEM)] * 4,
        out_specs=pl.BlockSpec(memory_space=pltpu.VMEM),
        scratch_shapes=[
            pltpu.VMEM((NC, RC, HD), BF16),
            pltpu.VMEM((NC, RC, HD), F32),
            pltpu.VMEM((NC, RC, HD), BF16),
            pltpu.VMEM((2, NC, RC, HD), BF16),
            pltpu.SemaphoreType.DMA((2, NC)),
            pltpu.SemaphoreType.DMA((2, NC)),
        ],
        compiler_params=pltpu.CompilerParams(collective_id=0),
    )(x16, Wq, KV16, Wo)


# device time: 14538 ns/iter; 1.0587x vs baseline; 1.0587x over previous
import jax
import jax.numpy as jnp
from jax import lax
from jax.experimental import pallas as pl
from jax.experimental.pallas import tpu as pltpu

N_DEV = 4
B, Sq, Skv, Hq, Dh = 2, 256, 1024, 4, 64
HD = Hq * Dh
D = 512
S_SH = Skv // N_DEV
NCH = 2
SQ_C = Sq // NCH
NC = B * NCH
RC = SQ_C + 8
F32 = jnp.float32
BF16 = jnp.bfloat16


def kernel(x, Wq, K_ext, V_ext, Wo):
    K2 = K_ext.reshape(B, S_SH, HD)
    V2 = V_ext.reshape(B, S_SH, HD)

    def body(x_ref, wq_ref, k_ref, v_ref, wo_ref, out_ref,
             pbuf, abuf, abuf_bf, rbuf, csend, crecv):
        my = lax.axis_index("i")
        left = lax.rem(my + N_DEV - 1, N_DEV)
        right = lax.rem(my + 1, N_DEV)
        p1 = my ^ 1
        p2 = 3 - my

        barrier = pltpu.get_barrier_semaphore()
        for nbr in (left, right):
            pl.semaphore_signal(barrier, inc=1, device_id=(nbr,),
                                device_id_type=pl.DeviceIdType.MESH)

        koff = my * S_SH
        wq16 = wq_ref[...].astype(BF16)
        wo16 = wo_ref[...].astype(BF16)

        def rdma(stage, c, src, target):
            return pltpu.make_async_remote_copy(
                src_ref=src.at[c], dst_ref=rbuf.at[stage, c],
                send_sem=csend.at[stage, c], recv_sem=crecv.at[stage, c],
                device_id=(target,), device_id_type=pl.DeviceIdType.MESH)

        s1 = [None] * NC
        s2 = [None] * NC
        first = True
        for b in range(B):
            k16 = k_ref[b].astype(BF16)
            v16 = v_ref[b].astype(BF16)
            for half in range(NCH):
                c = b * NCH + half
                r0 = half * SQ_C
                qi = lax.broadcasted_iota(jnp.int32, (SQ_C, S_SH), 0) + r0
                kig = lax.broadcasted_iota(jnp.int32, (SQ_C, S_SH), 1) + koff
                mask = (jnp.abs(qi - kig) <= 128) | (kig < 32) | (qi < 32)

                q_c = jnp.dot(
                    x_ref[b, r0:r0 + SQ_C, :].astype(BF16), wq16,
                    preferred_element_type=F32).astype(BF16)
                lcols = []
                for h in range(Hq):
                    qh = q_c[:, h * Dh:(h + 1) * Dh]
                    kh = k16[:, h * Dh:(h + 1) * Dh]
                    s = lax.dot_general(
                        qh, kh, (((1,), (1,)), ((), ())),
                        preferred_element_type=F32) * 0.125
                    w = jnp.where(mask, jnp.exp(s), 0.0)
                    vh = v16[:, h * Dh:(h + 1) * Dh]
                    pbuf[c, :SQ_C, h * Dh:(h + 1) * Dh] = jnp.dot(
                        w.astype(BF16), vh,
                        preferred_element_type=F32).astype(BF16)
                    lcols.append(jnp.sum(w, axis=1, keepdims=True))
                l_t = jnp.transpose(
                    jnp.concatenate(
                        lcols + [jnp.zeros((SQ_C, 8 - Hq), F32)],
                        axis=1))
                pbuf[c, SQ_C:, :] = jnp.concatenate(
                    [l_t, jnp.zeros((8, HD - SQ_C), F32)],
                    axis=1).astype(BF16)
                if first:
                    pl.semaphore_wait(barrier, 2)
                    first = False
                s1[c] = rdma(0, c, pbuf, p1)
                s1[c].start()

        for c in range(NC):
            s1[c].wait()
            abuf[c] = pbuf[c].astype(F32) + rbuf[0, c].astype(F32)
            abuf_bf[c] = abuf[c].astype(BF16)
            s2[c] = rdma(1, c, abuf_bf, p2)
            s2[c].start()

        for c in range(NC):
            b, half = divmod(c, NCH)
            s2[c].wait()
            tot = abuf[c] + rbuf[1, c].astype(F32)
            ctx = tot[:SQ_C, :]
            l_c = jnp.transpose(tot[SQ_C:, :SQ_C])
            rcp = 1.0 / l_c
            parts = []
            for h in range(Hq):
                parts.append(ctx[:, h * Dh:(h + 1) * Dh] * rcp[:, h:h + 1])
            ctx_n = jnp.concatenate(parts, axis=1)
            out_ref[b, half * SQ_C:(half + 1) * SQ_C, :] = jnp.dot(
                ctx_n.astype(BF16), wo16, preferred_element_type=F32)

    return pl.pallas_call(
        body,
        out_shape=jax.ShapeDtypeStruct((B, Sq, D), jnp.float32),
        in_specs=[pl.BlockSpec(memory_space=pltpu.VMEM)] * 5,
        out_specs=pl.BlockSpec(memory_space=pltpu.VMEM),
        scratch_shapes=[
            pltpu.VMEM((NC, RC, HD), BF16),
            pltpu.VMEM((NC, RC, HD), F32),
            pltpu.VMEM((NC, RC, HD), BF16),
            pltpu.VMEM((2, NC, RC, HD), BF16),
            pltpu.SemaphoreType.DMA((2, NC)),
            pltpu.SemaphoreType.DMA((2, NC)),
        ],
        compiler_params=pltpu.CompilerParams(collective_id=0),
    )(x, Wq, K2, V2, Wo)
